# baseline (speedup 1.0000x reference)
import jax
import jax.numpy as jnp
from jax import lax
from jax.experimental import pallas as pl

EMBED_DIM = 64
NUM_CONCEPTS = 36
BR = 1024


def _tc_body(idx_ref, table_ref, out_ref):
    br, ncol = idx_ref.shape
    table = table_ref[...]
    idx = idx_ref[...]
    for j in range(ncol):
        col = idx[:, j:j + 1]
        classes = lax.broadcasted_iota(jnp.int32, (br, NUM_CONCEPTS), 1)
        onehot = (col == classes).astype(jnp.float32)
        rows = jnp.dot(onehot, table, preferred_element_type=jnp.float32)
        out_ref[:, j * EMBED_DIM:(j + 1) * EMBED_DIM] = rows


def kernel(concept_idx, concepts_weight):
    n, ncol = concept_idx.shape
    idx = concept_idx.astype(jnp.int32)
    grid = n // BR
    out = pl.pallas_call(
        _tc_body,
        grid=(grid,),
        in_specs=[
            pl.BlockSpec((BR, ncol), lambda i: (i, 0)),
            pl.BlockSpec((NUM_CONCEPTS, EMBED_DIM), lambda i: (0, 0)),
        ],
        out_specs=pl.BlockSpec((BR, ncol * EMBED_DIM), lambda i: (i, 0)),
        out_shape=jax.ShapeDtypeStruct((n, ncol * EMBED_DIM), jnp.float32),
    )(idx, concepts_weight.astype(jnp.float32))
    return out.reshape(n, ncol, EMBED_DIM)


# speedup vs baseline: 1.0409x; 1.0409x over previous
import jax
import jax.numpy as jnp
from jax import lax
from jax.experimental import pallas as pl

EMBED_DIM = 64
NUM_CONCEPTS = 36
BR = 2048


def _tc_body(idx_ref, table_ref, out_ref):
    br, ncol = idx_ref.shape
    table = table_ref[...].astype(jnp.bfloat16)
    idx = idx_ref[...]
    for j in range(ncol):
        col = idx[:, j:j + 1]
        classes = lax.broadcasted_iota(jnp.int32, (br, NUM_CONCEPTS), 1)
        onehot = (col == classes).astype(jnp.bfloat16)
        rows = jnp.dot(onehot, table, preferred_element_type=jnp.float32)
        out_ref[:, j * EMBED_DIM:(j + 1) * EMBED_DIM] = rows


def kernel(concept_idx, concepts_weight):
    n, ncol = concept_idx.shape
    idx = concept_idx.astype(jnp.int32)
    grid = n // BR
    out = pl.pallas_call(
        _tc_body,
        grid=(grid,),
        in_specs=[
            pl.BlockSpec((BR, ncol), lambda i: (i, 0)),
            pl.BlockSpec((NUM_CONCEPTS, EMBED_DIM), lambda i: (0, 0)),
        ],
        out_specs=pl.BlockSpec((BR, ncol * EMBED_DIM), lambda i: (i, 0)),
        out_shape=jax.ShapeDtypeStruct((n, ncol * EMBED_DIM), jnp.float32),
    )(idx, concepts_weight.astype(jnp.float32))
    return out.reshape(n, ncol, EMBED_DIM)
